# Initial kernel scaffold; baseline (speedup 1.0000x reference)
import jax
import jax.numpy as jnp
from jax.experimental import pallas as pl


def kernel(input, word_embed_weight):
    raise NotImplementedError("write your pallas kernel here")


# speedup vs baseline: 6.1501x; 6.1501x over previous
import functools

import jax
import jax.numpy as jnp
from jax import lax
from jax.experimental import pallas as pl
from jax.experimental.pallas import tpu as pltpu
from jax.experimental.pallas import tpu_sc as plsc

BATCH = 16384
HIST = 50
D = 64
B = BATCH * HIST
NW = 32
BPW = B // NW
CHUNK = 1280
NCHUNK = BPW // CHUNK

_mesh = plsc.VectorSubcoreMesh(core_axis_name="c", subcore_axis_name="s")


@functools.partial(
    pl.kernel,
    mesh=_mesh,
    out_type=jax.ShapeDtypeStruct((B, D), jnp.float32),
    compiler_params=pltpu.CompilerParams(use_tc_tiling_on_sc=False),
    scratch_types=[
        pltpu.VMEM((CHUNK,), jnp.int32),
        pltpu.VMEM((CHUNK, D), jnp.float32),
        pltpu.SemaphoreType.DMA,
    ],
)
def _sc_gather(table_hbm, idx_hbm, out_hbm, idx_v, rows_v, sem):
    wid = lax.axis_index("s") * 2 + lax.axis_index("c")
    base = wid * BPW

    def body(c, carry):
        off = base + c * CHUNK
        pltpu.sync_copy(idx_hbm.at[pl.ds(off, CHUNK)], idx_v)
        pltpu.async_copy(table_hbm.at[idx_v], rows_v, sem).wait()
        pltpu.sync_copy(rows_v, out_hbm.at[pl.ds(off, CHUNK)])
        return carry

    lax.fori_loop(0, NCHUNK, body, 0)


def kernel(input, word_embed_weight):
    idx_flat = input.reshape(B)
    out = _sc_gather(word_embed_weight, idx_flat)
    return out.reshape(BATCH, HIST, D)


# speedup vs baseline: 6.2426x; 1.0150x over previous
import functools

import jax
import jax.numpy as jnp
from jax import lax
from jax.experimental import pallas as pl
from jax.experimental.pallas import tpu as pltpu
from jax.experimental.pallas import tpu_sc as plsc

BATCH = 16384
HIST = 50
D = 64
B = BATCH * HIST
NW = 32
BPW = B // NW
CHUNK = 640
NBUF = 2
NCHUNK = BPW // CHUNK

_mesh = plsc.VectorSubcoreMesh(core_axis_name="c", subcore_axis_name="s")


@functools.partial(
    pl.kernel,
    mesh=_mesh,
    out_type=jax.ShapeDtypeStruct((B, D), jnp.float32),
    compiler_params=pltpu.CompilerParams(use_tc_tiling_on_sc=False),
    scratch_types=[
        pltpu.VMEM((NCHUNK, CHUNK), jnp.int32),
        pltpu.VMEM((CHUNK, D), jnp.float32),
        pltpu.VMEM((CHUNK, D), jnp.float32),
        pltpu.SemaphoreType.DMA,
        pltpu.SemaphoreType.DMA,
        pltpu.SemaphoreType.DMA,
        pltpu.SemaphoreType.DMA,
    ],
)
def _sc_gather(table_hbm, idx_hbm, out_hbm, idx_v, rows0, rows1, g0, g1, s0, s1):
    rows = (rows0, rows1)
    gsem = (g0, g1)
    ssem = (s0, s1)
    wid = lax.axis_index("s") * 2 + lax.axis_index("c")
    base = wid * BPW

    pltpu.sync_copy(idx_hbm.at[wid], idx_v)

    def gather(c, b):
        return pltpu.make_async_copy(
            table_hbm.at[idx_v.at[c]], rows[b], gsem[b])

    def store(c, b):
        return pltpu.make_async_copy(
            rows[b], out_hbm.at[pl.ds(base + c * CHUNK, CHUNK)], ssem[b])

    gather(0, 0).start()
    gather(1, 1).start()
    gather(0, 0).wait()
    store(0, 0).start()

    def outer(g, carry):
        for bi in range(NBUF):
            c = g + bi
            store(c - NBUF, bi).wait()
            gather(c, bi).start()
            gather(c - 1, (bi + 1) % NBUF).wait()
            store(c - 1, (bi + 1) % NBUF).start()
        return carry

    lax.fori_loop(1, NCHUNK // NBUF, lambda i, cr: outer(i * NBUF, cr), 0)

    last = NCHUNK - 1
    gather(last, last % NBUF).wait()
    store(last, last % NBUF).start()
    store(last - 1, (last - 1) % NBUF).wait()
    store(last, last % NBUF).wait()


def kernel(input, word_embed_weight):
    idx = input.reshape(NW, NCHUNK, CHUNK)
    out = _sc_gather(word_embed_weight, idx)
    return out.reshape(BATCH, HIST, D)
